# Initial kernel scaffold; baseline (speedup 1.0000x reference)
import jax
import jax.numpy as jnp
from jax.experimental import pallas as pl


def kernel(bbox, position_ids, x_emb, y_emb, h_emb, w_emb, box_pos_emb, W, b, gamma, beta):
    raise NotImplementedError("write your pallas kernel here")


# speedup vs baseline: 1.7609x; 1.7609x over previous
import functools

import jax
import jax.numpy as jnp
from jax import lax
from jax.experimental import pallas as pl
from jax.experimental.pallas import tpu as pltpu
from jax.experimental.pallas import tpu_sc as plsc

B, S = 4, 2048
NTOK = B * S
D6 = 128
DOUT = 192
NSEG = DOUT // 16
NJ = 7
EPS = 1e-12

NC, NS = 2, 16
NW = NC * NS
SPAN = NTOK // NW
CH = 64
NCH = SPAN // CH


def _tc_table_body(x_ref, y_ref, h_ref, w_ref, bp_ref, w_mat_ref, b_ref, o_ref):
    f32 = jnp.float32
    o_ref[0:1024, :] = jnp.dot(x_ref[...], w_mat_ref[0:128, :], preferred_element_type=f32)
    o_ref[1024:2048, :] = jnp.dot(y_ref[...], w_mat_ref[128:256, :], preferred_element_type=f32)
    o_ref[2048:3072, :] = jnp.dot(x_ref[...], w_mat_ref[256:384, :], preferred_element_type=f32)
    o_ref[3072:4096, :] = jnp.dot(y_ref[...], w_mat_ref[384:512, :], preferred_element_type=f32)
    o_ref[4096:5120, :] = jnp.dot(h_ref[...], w_mat_ref[512:640, :], preferred_element_type=f32)
    o_ref[5120:6144, :] = jnp.dot(w_ref[...], w_mat_ref[640:768, :], preferred_element_type=f32)
    o_ref[6144:8192, :] = bp_ref[...] + b_ref[...]


def _build_table(x_emb, y_emb, h_emb, w_emb, box_pos_emb, w_mat, b):
    return pl.pallas_call(
        _tc_table_body,
        out_shape=jax.ShapeDtypeStruct((8192, DOUT), jnp.float32),
    )(x_emb, y_emb, h_emb, w_emb, box_pos_emb, w_mat, b.reshape(1, DOUT))


def _sc_body(t_hbm, bbox_hbm, pos_hbm, gam_hbm, bet_hbm, out_hbm,
             bbox_v, pos_v, idx_v, rows_v, obuf_v, gam_v, bet_v, sem):
    wid = lax.axis_index("s") * NC + lax.axis_index("c")
    pltpu.sync_copy(gam_hbm, gam_v)
    pltpu.sync_copy(bet_hbm, bet_v)

    def chunk_body(ch, carry):
        base = wid * SPAN + ch * CH
        for c in range(4):
            pltpu.sync_copy(bbox_hbm.at[c, pl.ds(base, CH)], bbox_v.at[c])
        pltpu.sync_copy(pos_hbm.at[pl.ds(base, CH)], pos_v)
        for g in range(CH // 16):
            sl = pl.ds(g * 16, 16)
            b0 = bbox_v[0, sl]
            b1 = bbox_v[1, sl]
            b2 = bbox_v[2, sl]
            b3 = bbox_v[3, sl]
            p = pos_v[sl]
            idx_v[0, sl] = b0
            idx_v[1, sl] = b1 + 1024
            idx_v[2, sl] = b2 + 2048
            idx_v[3, sl] = b3 + 3072
            idx_v[4, sl] = (b3 - b1) + 4096
            idx_v[5, sl] = (b2 - b0) + 5120
            idx_v[6, sl] = p + 6144
        for j in range(NJ):
            pltpu.async_copy(t_hbm.at[idx_v.at[j]], rows_v.at[j], sem).wait()
        def tok_body(t, c2):
            s = jnp.zeros((16,), jnp.float32)
            ss = jnp.zeros((16,), jnp.float32)
            for c in range(NSEG):
                csl = pl.ds(c * 16, 16)
                a = rows_v[0, t, csl]
                for j in range(1, NJ):
                    a = a + rows_v[j, t, csl]
                obuf_v[t, csl] = a
                s = s + a
                ss = ss + a * a
            mu_v = jnp.full((16,), jnp.sum(s), jnp.float32) * (1.0 / DOUT)
            var_v = jnp.full((16,), jnp.sum(ss), jnp.float32) * (1.0 / DOUT) - mu_v * mu_v
            v = var_v + EPS
            yi = jnp.full((16,), 0x5F3759DF, jnp.int32) - lax.shift_right_logical(
                plsc.bitcast(v, jnp.int32), jnp.full((16,), 1, jnp.int32))
            r = plsc.bitcast(yi, jnp.float32)
            for _ in range(3):
                r = r * (1.5 - 0.5 * v * r * r)
            for c in range(NSEG):
                csl = pl.ds(c * 16, 16)
                a = obuf_v[t, csl]
                obuf_v[t, csl] = (a - mu_v) * r * gam_v[csl] + bet_v[csl]
            return c2
        lax.fori_loop(0, CH, tok_body, 0)
        pltpu.sync_copy(obuf_v, out_hbm.at[pl.ds(base, CH), :])
        return carry
    lax.fori_loop(0, NCH, chunk_body, 0)


@functools.partial(
    pl.kernel,
    out_type=jax.ShapeDtypeStruct((NTOK, DOUT), jnp.float32),
    mesh=plsc.VectorSubcoreMesh(core_axis_name="c", subcore_axis_name="s",
                                num_cores=NC, num_subcores=NS),
    compiler_params=pltpu.CompilerParams(needs_layout_passes=False,
                                         use_tc_tiling_on_sc=False),
    scratch_types=[
        pltpu.VMEM((4, CH), jnp.int32),
        pltpu.VMEM((CH,), jnp.int32),
        pltpu.VMEM((NJ, CH), jnp.int32),
        pltpu.VMEM((NJ, CH, DOUT), jnp.float32),
        pltpu.VMEM((CH, DOUT), jnp.float32),
        pltpu.VMEM((DOUT,), jnp.float32),
        pltpu.VMEM((DOUT,), jnp.float32),
        pltpu.SemaphoreType.DMA,
    ],
)
def _sc_gather_ln(t_hbm, bbox_hbm, pos_hbm, gam_hbm, bet_hbm, out_hbm, *rest):
    _sc_body(t_hbm, bbox_hbm, pos_hbm, gam_hbm, bet_hbm, out_hbm, *rest)


def kernel(bbox, position_ids, x_emb, y_emb, h_emb, w_emb, box_pos_emb, W, b, gamma, beta):
    table = _build_table(x_emb, y_emb, h_emb, w_emb, box_pos_emb, W, b)
    bbox_flat = bbox.reshape(NTOK, 4).astype(jnp.int32).T
    pos_flat = position_ids.reshape(NTOK).astype(jnp.int32)
    out = _sc_gather_ln(table, bbox_flat, pos_flat, gamma, beta)
    return out.reshape(B, S, DOUT)


# speedup vs baseline: 2.1081x; 1.1971x over previous
import functools

import jax
import jax.numpy as jnp
from jax import lax
from jax.experimental import pallas as pl
from jax.experimental.pallas import tpu as pltpu
from jax.experimental.pallas import tpu_sc as plsc

B, S = 4, 2048
NTOK = B * S
D6 = 128
DOUT = 192
NSEG = DOUT // 16
NJ = 7
EPS = 1e-12

NC, NS = 2, 16
NW = NC * NS
SPAN = NTOK // NW
CH = 32
NCH = SPAN // CH


def _tc_table_body(x_ref, y_ref, h_ref, w_ref, bp_ref, w_mat_ref, b_ref, o_ref):
    f32 = jnp.float32
    o_ref[0:1024, :] = jnp.dot(x_ref[...], w_mat_ref[0:128, :], preferred_element_type=f32)
    o_ref[1024:2048, :] = jnp.dot(y_ref[...], w_mat_ref[128:256, :], preferred_element_type=f32)
    o_ref[2048:3072, :] = jnp.dot(x_ref[...], w_mat_ref[256:384, :], preferred_element_type=f32)
    o_ref[3072:4096, :] = jnp.dot(y_ref[...], w_mat_ref[384:512, :], preferred_element_type=f32)
    o_ref[4096:5120, :] = jnp.dot(h_ref[...], w_mat_ref[512:640, :], preferred_element_type=f32)
    o_ref[5120:6144, :] = jnp.dot(w_ref[...], w_mat_ref[640:768, :], preferred_element_type=f32)
    o_ref[6144:8192, :] = bp_ref[...] + b_ref[...]


def _build_table(x_emb, y_emb, h_emb, w_emb, box_pos_emb, w_mat, b):
    return pl.pallas_call(
        _tc_table_body,
        out_shape=jax.ShapeDtypeStruct((8192, DOUT), jnp.float32),
    )(x_emb, y_emb, h_emb, w_emb, box_pos_emb, w_mat, b.reshape(1, DOUT))


def _sc_body(t_hbm, bbox_hbm, pos_hbm, gam_hbm, bet_hbm, out_hbm,
             bbox_v, pos_v, idx_v, rows_v, obuf_v, gam_v, bet_v, sem0, sem1):
    sems = (sem0, sem1)
    wid = lax.axis_index("s") * NC + lax.axis_index("c")
    pltpu.sync_copy(gam_hbm, gam_v)
    pltpu.sync_copy(bet_hbm, bet_v)

    def fire(ch, bf):
        base = wid * SPAN + ch * CH
        pltpu.sync_copy(bbox_hbm.at[pl.ds(base * 4, CH * 4)], bbox_v.at[bf])
        pltpu.sync_copy(pos_hbm.at[pl.ds(base, CH)], pos_v.at[bf])
        for g in range(CH // 16):
            row4 = (lax.iota(jnp.int32, 16) + g * 16) * 4
            b0 = plsc.load_gather(bbox_v.at[bf], [row4])
            b1 = plsc.load_gather(bbox_v.at[bf], [row4 + 1])
            b2 = plsc.load_gather(bbox_v.at[bf], [row4 + 2])
            b3 = plsc.load_gather(bbox_v.at[bf], [row4 + 3])
            sl = pl.ds(g * 16, 16)
            p = pos_v[bf, sl]
            idx_v[bf, 0, sl] = b0
            idx_v[bf, 1, sl] = b1 + 1024
            idx_v[bf, 2, sl] = b2 + 2048
            idx_v[bf, 3, sl] = b3 + 3072
            idx_v[bf, 4, sl] = (b3 - b1) + 4096
            idx_v[bf, 5, sl] = (b2 - b0) + 5120
            idx_v[bf, 6, sl] = p + 6144
        return [pltpu.async_copy(t_hbm.at[idx_v.at[bf, j]], rows_v.at[bf, j],
                                 sems[bf]) for j in range(NJ)]

    def compute(ch, bf):
        base = wid * SPAN + ch * CH

        def tok_body(t, c2):
            s = jnp.zeros((16,), jnp.float32)
            ss = jnp.zeros((16,), jnp.float32)
            for c in range(NSEG):
                csl = pl.ds(c * 16, 16)
                a = rows_v[bf, 0, t, csl]
                for j in range(1, NJ):
                    a = a + rows_v[bf, j, t, csl]
                obuf_v[t, csl] = a
                s = s + a
                ss = ss + a * a
            mu_v = jnp.full((16,), jnp.sum(s), jnp.float32) * (1.0 / DOUT)
            var_v = jnp.full((16,), jnp.sum(ss), jnp.float32) * (1.0 / DOUT) - mu_v * mu_v
            v = var_v + EPS
            yi = jnp.full((16,), 0x5F3759DF, jnp.int32) - lax.shift_right_logical(
                plsc.bitcast(v, jnp.int32), jnp.full((16,), 1, jnp.int32))
            r = plsc.bitcast(yi, jnp.float32)
            for _ in range(3):
                r = r * (1.5 - 0.5 * v * r * r)
            for c in range(NSEG):
                csl = pl.ds(c * 16, 16)
                a = obuf_v[t, csl]
                obuf_v[t, csl] = (a - mu_v) * r * gam_v[csl] + bet_v[csl]
            return c2
        lax.fori_loop(0, CH, tok_body, 0)
        pltpu.sync_copy(obuf_v, out_hbm.at[pl.ds(base, CH), :])

    handles = {0: fire(0, 0), 1: fire(1, 1)}
    for ch in range(NCH):
        bf = ch % 2
        for h in handles.pop(ch):
            h.wait()
        compute(ch, bf)
        if ch + 2 < NCH:
            handles[ch + 2] = fire(ch + 2, bf)


@functools.partial(
    pl.kernel,
    out_type=jax.ShapeDtypeStruct((NTOK, DOUT), jnp.float32),
    mesh=plsc.VectorSubcoreMesh(core_axis_name="c", subcore_axis_name="s",
                                num_cores=NC, num_subcores=NS),
    compiler_params=pltpu.CompilerParams(needs_layout_passes=False,
                                         use_tc_tiling_on_sc=False),
    scratch_types=[
        pltpu.VMEM((2, CH * 4), jnp.int32),
        pltpu.VMEM((2, CH), jnp.int32),
        pltpu.VMEM((2, NJ, CH), jnp.int32),
        pltpu.VMEM((2, NJ, CH, DOUT), jnp.float32),
        pltpu.VMEM((CH, DOUT), jnp.float32),
        pltpu.VMEM((DOUT,), jnp.float32),
        pltpu.VMEM((DOUT,), jnp.float32),
        pltpu.SemaphoreType.DMA,
        pltpu.SemaphoreType.DMA,
    ],
)
def _sc_gather_ln(t_hbm, bbox_hbm, pos_hbm, gam_hbm, bet_hbm, out_hbm, *rest):
    _sc_body(t_hbm, bbox_hbm, pos_hbm, gam_hbm, bet_hbm, out_hbm, *rest)


def kernel(bbox, position_ids, x_emb, y_emb, h_emb, w_emb, box_pos_emb, W, b, gamma, beta):
    table = _build_table(x_emb, y_emb, h_emb, w_emb, box_pos_emb, W, b)
    bbox_flat = bbox.reshape(NTOK * 4).astype(jnp.int32)
    pos_flat = position_ids.reshape(NTOK).astype(jnp.int32)
    out = _sc_gather_ln(table, bbox_flat, pos_flat, gamma, beta)
    return out.reshape(B, S, DOUT)


# speedup vs baseline: 2.4538x; 1.1640x over previous
import functools

import jax
import jax.numpy as jnp
from jax import lax
from jax.experimental import pallas as pl
from jax.experimental.pallas import tpu as pltpu
from jax.experimental.pallas import tpu_sc as plsc

B, S = 4, 2048
NTOK = B * S
D6 = 128
DOUT = 192
NSEG = DOUT // 16
NJ = 7
EPS = 1e-12

NC, NS = 2, 16
NW = NC * NS
SPAN = NTOK // NW
CH = 32
NCH = SPAN // CH


def _tc_table_body(x_ref, y_ref, h_ref, w_ref, bp_ref, w_mat_ref, b_ref, o_ref):
    f32 = jnp.float32
    o_ref[0:1024, :] = jnp.dot(x_ref[...], w_mat_ref[0:128, :], preferred_element_type=f32)
    o_ref[1024:2048, :] = jnp.dot(y_ref[...], w_mat_ref[128:256, :], preferred_element_type=f32)
    o_ref[2048:3072, :] = jnp.dot(x_ref[...], w_mat_ref[256:384, :], preferred_element_type=f32)
    o_ref[3072:4096, :] = jnp.dot(y_ref[...], w_mat_ref[384:512, :], preferred_element_type=f32)
    o_ref[4096:5120, :] = jnp.dot(h_ref[...], w_mat_ref[512:640, :], preferred_element_type=f32)
    o_ref[5120:6144, :] = jnp.dot(w_ref[...], w_mat_ref[640:768, :], preferred_element_type=f32)
    o_ref[6144:8192, :] = bp_ref[...] + b_ref[...]


def _build_table(x_emb, y_emb, h_emb, w_emb, box_pos_emb, w_mat, b):
    return pl.pallas_call(
        _tc_table_body,
        out_shape=jax.ShapeDtypeStruct((8192, DOUT), jnp.float32),
    )(x_emb, y_emb, h_emb, w_emb, box_pos_emb, w_mat, b.reshape(1, DOUT))


def _sc_body(t_hbm, bbox_hbm, pos_hbm, gam_hbm, bet_hbm, out_hbm,
             bbox_v, pos_v, idx_v, acc_v, obuf_v, gam_v, bet_v, sem0, sem1):
    sems = (sem0, sem1)
    wid = lax.axis_index("s") * NC + lax.axis_index("c")
    pltpu.sync_copy(gam_hbm, gam_v)
    pltpu.sync_copy(bet_hbm, bet_v)

    def fire(ch, bf):
        base = wid * SPAN + ch * CH
        pltpu.sync_copy(bbox_hbm.at[pl.ds(base * 4, CH * 4)], bbox_v.at[bf])
        pltpu.sync_copy(pos_hbm.at[pl.ds(base, CH)], pos_v.at[bf])
        for g in range(CH // 16):
            row4 = (lax.iota(jnp.int32, 16) + g * 16) * 4
            b0 = plsc.load_gather(bbox_v.at[bf], [row4])
            b1 = plsc.load_gather(bbox_v.at[bf], [row4 + 1])
            b2 = plsc.load_gather(bbox_v.at[bf], [row4 + 2])
            b3 = plsc.load_gather(bbox_v.at[bf], [row4 + 3])
            sl = pl.ds(g * 16, 16)
            p = pos_v[bf, sl]
            idx_v[bf, 0, sl] = b0
            idx_v[bf, 1, sl] = b1 + 1024
            idx_v[bf, 2, sl] = b2 + 2048
            idx_v[bf, 3, sl] = b3 + 3072
            idx_v[bf, 4, sl] = (b3 - b1) + 4096
            idx_v[bf, 5, sl] = (b2 - b0) + 5120
            idx_v[bf, 6, sl] = p + 6144
        zero = jnp.zeros((16,), jnp.float32)

        def zero_body(t, c2):
            for c in range(NSEG):
                acc_v[bf, t, pl.ds(c * 16, 16)] = zero
            return c2
        lax.fori_loop(0, CH, zero_body, 0)
        return [pltpu.async_copy(t_hbm.at[idx_v.at[bf, j]], acc_v.at[bf],
                                 sems[bf], add=True) for j in range(NJ)]

    def compute(ch, bf):
        base = wid * SPAN + ch * CH

        def tok_body(t, c2):
            s = jnp.zeros((16,), jnp.float32)
            ss = jnp.zeros((16,), jnp.float32)
            for c in range(NSEG):
                csl = pl.ds(c * 16, 16)
                a = acc_v[bf, t, csl]
                s = s + a
                ss = ss + a * a
            mu_v = jnp.full((16,), jnp.sum(s), jnp.float32) * (1.0 / DOUT)
            var_v = jnp.full((16,), jnp.sum(ss), jnp.float32) * (1.0 / DOUT) - mu_v * mu_v
            v = var_v + EPS
            yi = jnp.full((16,), 0x5F3759DF, jnp.int32) - lax.shift_right_logical(
                plsc.bitcast(v, jnp.int32), jnp.full((16,), 1, jnp.int32))
            r = plsc.bitcast(yi, jnp.float32)
            for _ in range(3):
                r = r * (1.5 - 0.5 * v * r * r)
            for c in range(NSEG):
                csl = pl.ds(c * 16, 16)
                a = acc_v[bf, t, csl]
                obuf_v[t, csl] = (a - mu_v) * r * gam_v[csl] + bet_v[csl]
            return c2
        lax.fori_loop(0, CH, tok_body, 0)
        pltpu.sync_copy(obuf_v, out_hbm.at[pl.ds(base, CH), :])

    handles = {0: fire(0, 0), 1: fire(1, 1)}
    for ch in range(NCH):
        bf = ch % 2
        for h in handles.pop(ch):
            h.wait()
        compute(ch, bf)
        if ch + 2 < NCH:
            handles[ch + 2] = fire(ch + 2, bf)


@functools.partial(
    pl.kernel,
    out_type=jax.ShapeDtypeStruct((NTOK, DOUT), jnp.float32),
    mesh=plsc.VectorSubcoreMesh(core_axis_name="c", subcore_axis_name="s",
                                num_cores=NC, num_subcores=NS),
    compiler_params=pltpu.CompilerParams(needs_layout_passes=False,
                                         use_tc_tiling_on_sc=False),
    scratch_types=[
        pltpu.VMEM((2, CH * 4), jnp.int32),
        pltpu.VMEM((2, CH), jnp.int32),
        pltpu.VMEM((2, NJ, CH), jnp.int32),
        pltpu.VMEM((2, CH, DOUT), jnp.float32),
        pltpu.VMEM((CH, DOUT), jnp.float32),
        pltpu.VMEM((DOUT,), jnp.float32),
        pltpu.VMEM((DOUT,), jnp.float32),
        pltpu.SemaphoreType.DMA,
        pltpu.SemaphoreType.DMA,
    ],
)
def _sc_gather_ln(t_hbm, bbox_hbm, pos_hbm, gam_hbm, bet_hbm, out_hbm, *rest):
    _sc_body(t_hbm, bbox_hbm, pos_hbm, gam_hbm, bet_hbm, out_hbm, *rest)


def kernel(bbox, position_ids, x_emb, y_emb, h_emb, w_emb, box_pos_emb, W, b, gamma, beta):
    table = _build_table(x_emb, y_emb, h_emb, w_emb, box_pos_emb, W, b)
    bbox_flat = bbox.reshape(NTOK * 4).astype(jnp.int32)
    pos_flat = position_ids.reshape(NTOK).astype(jnp.int32)
    out = _sc_gather_ln(table, bbox_flat, pos_flat, gamma, beta)
    return out.reshape(B, S, DOUT)


# speedup vs baseline: 2.7108x; 1.1047x over previous
import functools

import jax
import jax.numpy as jnp
from jax import lax
from jax.experimental import pallas as pl
from jax.experimental.pallas import tpu as pltpu
from jax.experimental.pallas import tpu_sc as plsc

B, S = 4, 2048
NTOK = B * S
DOUT = 192
NSEG = DOUT // 16
NSEG_L = 8
NJ = 7
EPS = 1e-12

NC, NS = 2, 16
NW = NC * NS
SPAN = NTOK // NW
CH = 64
NCH = SPAN // CH


def _tc_table_body(x_ref, y_ref, h_ref, w_ref, bp_ref, w_mat_ref, b_ref,
                   tl_ref, tr_ref):
    f32 = jnp.float32
    embs = (x_ref, y_ref, x_ref, y_ref, h_ref, w_ref)
    for i, e in enumerate(embs):
        d = jnp.dot(e[...], w_mat_ref[i * 128:(i + 1) * 128, :],
                    preferred_element_type=f32)
        tl_ref[i * 1024:(i + 1) * 1024, :] = d[:, 0:128]
        tr_ref[i * 1024:(i + 1) * 1024, 0:64] = d[:, 128:192]
    bp = bp_ref[...] + b_ref[...]
    tl_ref[6144:8192, :] = bp[:, 0:128]
    tr_ref[6144:8192, 0:64] = bp[:, 128:192]
    tr_ref[:, 64:128] = jnp.zeros((NTOK, 64), f32)


def _build_tables(x_emb, y_emb, h_emb, w_emb, box_pos_emb, w_mat, b):
    return pl.pallas_call(
        _tc_table_body,
        out_shape=(jax.ShapeDtypeStruct((NTOK, 128), jnp.float32),
                   jax.ShapeDtypeStruct((NTOK, 128), jnp.float32)),
    )(x_emb, y_emb, h_emb, w_emb, box_pos_emb, w_mat, b.reshape(1, DOUT))


def _sc_body(tl_hbm, tr_hbm, bbox_hbm, pos_hbm, gam_hbm, bet_hbm,
             outl_hbm, outr_hbm,
             bb0, bb1, pp0, pp1, ix0, ix1, al0, al1, ar0, ar1,
             obl_v, obr_v, gam_v, bet_v, sem0, sem1):
    bbs, pps, ixs = (bb0, bb1), (pp0, pp1), (ix0, ix1)
    als, ars, sems = (al0, al1), (ar0, ar1), (sem0, sem1)
    wid = lax.axis_index("s") * NC + lax.axis_index("c")
    pltpu.sync_copy(gam_hbm, gam_v)
    pltpu.sync_copy(bet_hbm, bet_v)

    def fire(ch, bf):
        base = wid * SPAN + ch * CH
        bb, pp, ix = bbs[bf], pps[bf], ixs[bf]
        al, ar = als[bf], ars[bf]
        pltpu.sync_copy(bbox_hbm.at[pl.ds(base * 4, CH * 4)], bb)
        pltpu.sync_copy(pos_hbm.at[pl.ds(base, CH)], pp)
        for g in range(CH // 16):
            row4 = (lax.iota(jnp.int32, 16) + g * 16) * 4
            b0 = plsc.load_gather(bb, [row4])
            b1 = plsc.load_gather(bb, [row4 + 1])
            b2 = plsc.load_gather(bb, [row4 + 2])
            b3 = plsc.load_gather(bb, [row4 + 3])
            p = pp[pl.ds(g * 16, 16)]
            ix[pl.ds(0 * CH + g * 16, 16)] = b0
            ix[pl.ds(1 * CH + g * 16, 16)] = b1 + 1024
            ix[pl.ds(2 * CH + g * 16, 16)] = b2 + 2048
            ix[pl.ds(3 * CH + g * 16, 16)] = b3 + 3072
            ix[pl.ds(4 * CH + g * 16, 16)] = (b3 - b1) + 4096
            ix[pl.ds(5 * CH + g * 16, 16)] = (b2 - b0) + 5120
            ix[pl.ds(6 * CH + g * 16, 16)] = p + 6144
        zero = jnp.zeros((16,), jnp.float32)

        def zero_body(t, c2):
            for c in range(NSEG_L):
                al[t, pl.ds(c * 16, 16)] = zero
            for c in range(NSEG - NSEG_L):
                ar[t, pl.ds(c * 16, 16)] = zero
            return c2
        lax.fori_loop(0, CH, zero_body, 0)
        hs = []
        for j in range(NJ):
            isl = ix.at[pl.ds(j * CH, CH)]
            hs.append(pltpu.async_copy(tl_hbm.at[isl], al, sems[bf], add=True))
            hs.append(pltpu.async_copy(tr_hbm.at[isl], ar, sems[bf], add=True))
        return hs

    def compute(ch, bf):
        base = wid * SPAN + ch * CH
        al, ar = als[bf], ars[bf]

        def tok_body(t, c2):
            s = jnp.zeros((16,), jnp.float32)
            ss = jnp.zeros((16,), jnp.float32)
            for c in range(NSEG):
                if c < NSEG_L:
                    a = al[t, pl.ds(c * 16, 16)]
                else:
                    a = ar[t, pl.ds((c - NSEG_L) * 16, 16)]
                s = s + a
                ss = ss + a * a
            mu_v = jnp.full((16,), jnp.sum(s), jnp.float32) * (1.0 / DOUT)
            var_v = jnp.full((16,), jnp.sum(ss), jnp.float32) * (1.0 / DOUT) - mu_v * mu_v
            v = var_v + EPS
            yi = jnp.full((16,), 0x5F3759DF, jnp.int32) - lax.shift_right_logical(
                plsc.bitcast(v, jnp.int32), jnp.full((16,), 1, jnp.int32))
            r = plsc.bitcast(yi, jnp.float32)
            for _ in range(3):
                r = r * (1.5 - 0.5 * v * r * r)
            for c in range(NSEG):
                csl = pl.ds(c * 16, 16)
                if c < NSEG_L:
                    a = al[t, csl]
                    obl_v[t, csl] = (a - mu_v) * r * gam_v[csl] + bet_v[csl]
                else:
                    rsl = pl.ds((c - NSEG_L) * 16, 16)
                    a = ar[t, rsl]
                    obr_v[t, rsl] = (a - mu_v) * r * gam_v[csl] + bet_v[csl]
            return c2
        lax.fori_loop(0, CH, tok_body, 0)
        pltpu.sync_copy(obl_v, outl_hbm.at[pl.ds(base, CH), :])
        pltpu.sync_copy(obr_v, outr_hbm.at[pl.ds(base, CH), :])

    handles = {0: fire(0, 0), 1: fire(1, 1)}
    for ch in range(NCH):
        bf = ch % 2
        for h in handles.pop(ch):
            h.wait()
        compute(ch, bf)
        if ch + 2 < NCH:
            handles[ch + 2] = fire(ch + 2, bf)


@functools.partial(
    pl.kernel,
    out_type=(jax.ShapeDtypeStruct((NTOK, 128), jnp.float32),
              jax.ShapeDtypeStruct((NTOK, 128), jnp.float32)),
    mesh=plsc.VectorSubcoreMesh(core_axis_name="c", subcore_axis_name="s",
                                num_cores=NC, num_subcores=NS),
    compiler_params=pltpu.CompilerParams(needs_layout_passes=False,
                                         use_tc_tiling_on_sc=False),
    scratch_types=[
        pltpu.VMEM((CH * 4,), jnp.int32),
        pltpu.VMEM((CH * 4,), jnp.int32),
        pltpu.VMEM((CH,), jnp.int32),
        pltpu.VMEM((CH,), jnp.int32),
        pltpu.VMEM((NJ * CH,), jnp.int32),
        pltpu.VMEM((NJ * CH,), jnp.int32),
        pltpu.VMEM((CH, 128), jnp.float32),
        pltpu.VMEM((CH, 128), jnp.float32),
        pltpu.VMEM((CH, 128), jnp.float32),
        pltpu.VMEM((CH, 128), jnp.float32),
        pltpu.VMEM((CH, 128), jnp.float32),
        pltpu.VMEM((CH, 128), jnp.float32),
        pltpu.VMEM((DOUT,), jnp.float32),
        pltpu.VMEM((DOUT,), jnp.float32),
        pltpu.SemaphoreType.DMA,
        pltpu.SemaphoreType.DMA,
    ],
)
def _sc_gather_ln(tl_hbm, tr_hbm, bbox_hbm, pos_hbm, gam_hbm, bet_hbm,
                  outl_hbm, outr_hbm, *rest):
    _sc_body(tl_hbm, tr_hbm, bbox_hbm, pos_hbm, gam_hbm, bet_hbm,
             outl_hbm, outr_hbm, *rest)


def kernel(bbox, position_ids, x_emb, y_emb, h_emb, w_emb, box_pos_emb, W, b, gamma, beta):
    tl, tr = _build_tables(x_emb, y_emb, h_emb, w_emb, box_pos_emb, W, b)
    bbox_flat = bbox.reshape(NTOK * 4).astype(jnp.int32)
    pos_flat = position_ids.reshape(NTOK).astype(jnp.int32)
    outl, outr = _sc_gather_ln(tl, tr, bbox_flat, pos_flat, gamma, beta)
    out = jnp.concatenate([outl, outr[:, :64]], axis=-1)
    return out.reshape(B, S, DOUT)
